# baseline (speedup 1.0000x reference)
import functools

import jax
import jax.numpy as jnp
from jax.experimental import pallas as pl
from jax.experimental.pallas import tpu as pltpu

ALPHA = 0.7
TAU_WORD = 0.1


def _wsc_kernel(tgt_ref, mask_ref, in_ref, sim_ref, ml_ref, tot_ref, acc_ref):
    i = pl.program_id(0)
    n = pl.num_programs(0)

    @pl.when(i == 0)
    def _init():
        acc_ref[0] = 0.0
        acc_ref[1] = 0.0
        acc_ref[2] = 0.0
        acc_ref[3] = 0.0

    m = mask_ref[i]
    tgt = tgt_ref[i]
    row_in = in_ref[0, 0, :]
    smooth = jnp.exp((sim_ref[0, 0, :] - 1.0) * (1.0 / TAU_WORD))
    dot = jnp.sum(row_in * smooth)
    ssum = jnp.sum(smooth)
    col = jax.lax.broadcasted_iota(jnp.int32, row_in.shape, 0)
    tval = jnp.sum(jnp.where(col == tgt, row_in, 0.0))

    acc_ref[0] += -tval * m
    acc_ref[1] += m
    acc_ref[2] += -dot * m
    acc_ref[3] += ssum * m

    @pl.when(i == n - 1)
    def _fin():
        ml = acc_ref[0] / acc_ref[1]
        smooth_loss = acc_ref[2] / acc_ref[3]
        ml_ref[0] = ml
        tot_ref[0] = ALPHA * smooth_loss + (1.0 - ALPHA) * ml


@functools.partial(jax.jit, static_argnames=())
def _run(flat_in, flat_t, flat_m, Sim_Matrix):
    n, _, v = flat_in.shape
    grid_spec = pltpu.PrefetchScalarGridSpec(
        num_scalar_prefetch=2,
        grid=(n,),
        in_specs=[
            pl.BlockSpec((1, 1, v), lambda i, tgt, msk: (i, 0, 0)),
            pl.BlockSpec((1, 1, v), lambda i, tgt, msk: (tgt[i], 0, 0)),
        ],
        out_specs=[
            pl.BlockSpec(memory_space=pltpu.SMEM),
            pl.BlockSpec(memory_space=pltpu.SMEM),
        ],
        scratch_shapes=[pltpu.SMEM((4,), jnp.float32)],
    )
    ml, tot = pl.pallas_call(
        _wsc_kernel,
        grid_spec=grid_spec,
        out_shape=[
            jax.ShapeDtypeStruct((1,), jnp.float32),
            jax.ShapeDtypeStruct((1,), jnp.float32),
        ],
    )(flat_t, flat_m, flat_in, Sim_Matrix.reshape(v, 1, v))
    return ml[0], tot[0]


def kernel(input, target, mask, Sim_Matrix):
    b, t, v = input.shape
    flat_in = input.reshape(-1, 1, v)
    flat_t = target[:, :t].reshape(-1)
    flat_m = mask[:, :t].reshape(-1)
    return _run(flat_in, flat_t, flat_m, Sim_Matrix)


# speedup vs baseline: 2.8262x; 2.8262x over previous
import functools

import jax
import jax.numpy as jnp
from jax.experimental import pallas as pl
from jax.experimental.pallas import tpu as pltpu

ALPHA = 0.7
TAU_WORD = 0.1
K = 8
SUB = 8


def _wsc_kernel(tgt_ref, mask_ref, in_ref, *rest):
    sim_refs = rest[:K]
    ml_ref, tot_ref, smem_acc, pr_acc, ss_acc = rest[K:]
    i = pl.program_id(0)
    n = pl.num_programs(0)
    lanes = in_ref.shape[-1]

    @pl.when(i == 0)
    def _init():
        smem_acc[0] = 0.0
        smem_acc[1] = 0.0
        pr_acc[...] = jnp.zeros_like(pr_acc)
        ss_acc[...] = jnp.zeros_like(ss_acc)

    sub_iota = jax.lax.broadcasted_iota(jnp.int32, (SUB, lanes), 0)
    lane_iota = jax.lax.broadcasted_iota(jnp.int32, (SUB, lanes), 1)

    for k in range(K):
        m = mask_ref[i * K + k]
        tgt = tgt_ref[i * K + k]
        row_in = in_ref[k]
        smooth = jnp.exp((sim_refs[k][0] - 1.0) * (1.0 / TAU_WORD))
        t = smooth * m
        ss_acc[...] += t
        pr_acc[...] += row_in * t
        hit = (sub_iota == tgt // lanes) & (lane_iota == tgt % lanes)
        tval = jnp.sum(jnp.where(hit, row_in, 0.0))
        smem_acc[0] += -tval * m
        smem_acc[1] += m

    @pl.when(i == n - 1)
    def _fin():
        ml = smem_acc[0] / smem_acc[1]
        smooth_loss = -jnp.sum(pr_acc[...]) / jnp.sum(ss_acc[...])
        ml_ref[0] = ml
        tot_ref[0] = ALPHA * smooth_loss + (1.0 - ALPHA) * ml


@jax.jit
def _run(flat_in, flat_t, flat_m, sim3):
    n, sub, lanes = flat_in.shape
    grid_spec = pltpu.PrefetchScalarGridSpec(
        num_scalar_prefetch=2,
        grid=(n // K,),
        in_specs=[
            pl.BlockSpec((K, sub, lanes), lambda i, tgt, msk: (i, 0, 0)),
        ] + [
            pl.BlockSpec(
                (1, sub, lanes),
                functools.partial(
                    lambda i, tgt, msk, k: (tgt[i * K + k], 0, 0), k=k
                ),
            )
            for k in range(K)
        ],
        out_specs=[
            pl.BlockSpec(memory_space=pltpu.SMEM),
            pl.BlockSpec(memory_space=pltpu.SMEM),
        ],
        scratch_shapes=[
            pltpu.SMEM((2,), jnp.float32),
            pltpu.VMEM((sub, lanes), jnp.float32),
            pltpu.VMEM((sub, lanes), jnp.float32),
        ],
    )
    ml, tot = pl.pallas_call(
        _wsc_kernel,
        grid_spec=grid_spec,
        out_shape=[
            jax.ShapeDtypeStruct((1,), jnp.float32),
            jax.ShapeDtypeStruct((1,), jnp.float32),
        ],
    )(flat_t, flat_m, flat_in, *([sim3] * K))
    return ml[0], tot[0]


def kernel(input, target, mask, Sim_Matrix):
    b, t, v = input.shape
    flat_in = input.reshape(b * t, SUB, v // SUB)
    flat_t = target[:, :t].reshape(-1)
    flat_m = mask[:, :t].reshape(-1)
    sim3 = Sim_Matrix.reshape(v, SUB, v // SUB)
    return _run(flat_in, flat_t, flat_m, sim3)


# speedup vs baseline: 3.3918x; 1.2001x over previous
import functools

import jax
import jax.numpy as jnp
from jax.experimental import pallas as pl
from jax.experimental.pallas import tpu as pltpu

ALPHA = 0.7
TAU_WORD = 0.1
K = 32
SUB = 8


def _wsc_kernel(tgt_ref, mask_ref, in_ref, *rest):
    sim_refs = rest[:K]
    ml_ref, tot_ref, mask_acc, pr_acc, ss_acc, ml_acc = rest[K:]
    i = pl.program_id(0)
    n = pl.num_programs(0)
    lanes = in_ref.shape[-1]

    @pl.when(i == 0)
    def _init():
        mask_acc[0] = 0.0
        pr_acc[...] = jnp.zeros_like(pr_acc)
        ss_acc[...] = jnp.zeros_like(ss_acc)
        ml_acc[...] = jnp.zeros_like(ml_acc)

    flat_iota = (
        jax.lax.broadcasted_iota(jnp.int32, (SUB, lanes), 0) * lanes
        + jax.lax.broadcasted_iota(jnp.int32, (SUB, lanes), 1)
    )
    zeros = jnp.zeros((SUB, lanes), jnp.float32)
    pr_t = zeros
    ss_t = zeros
    ml_t = zeros
    msum = 0.0
    for k in range(K):
        m = mask_ref[i * K + k]
        tgt = tgt_ref[i * K + k]
        row_in = in_ref[k]
        smooth = jnp.exp((sim_refs[k][0] - 1.0) * (1.0 / TAU_WORD))
        t = smooth * m
        ss_t += t
        pr_t += row_in * t
        ml_t += jnp.where(flat_iota == tgt, row_in, 0.0) * m
        msum += m
    pr_acc[...] += pr_t
    ss_acc[...] += ss_t
    ml_acc[...] += ml_t
    mask_acc[0] += msum

    @pl.when(i == n - 1)
    def _fin():
        ml = -jnp.sum(ml_acc[...]) / mask_acc[0]
        smooth_loss = -jnp.sum(pr_acc[...]) / jnp.sum(ss_acc[...])
        ml_ref[0] = ml
        tot_ref[0] = ALPHA * smooth_loss + (1.0 - ALPHA) * ml


@jax.jit
def _run(flat_in, flat_t, flat_m, sim3):
    n, sub, lanes = flat_in.shape
    grid_spec = pltpu.PrefetchScalarGridSpec(
        num_scalar_prefetch=2,
        grid=(n // K,),
        in_specs=[
            pl.BlockSpec((K, sub, lanes), lambda i, tgt, msk: (i, 0, 0)),
        ] + [
            pl.BlockSpec(
                (1, sub, lanes),
                functools.partial(
                    lambda i, tgt, msk, k: (tgt[i * K + k], 0, 0), k=k
                ),
            )
            for k in range(K)
        ],
        out_specs=[
            pl.BlockSpec(memory_space=pltpu.SMEM),
            pl.BlockSpec(memory_space=pltpu.SMEM),
        ],
        scratch_shapes=[
            pltpu.SMEM((1,), jnp.float32),
            pltpu.VMEM((sub, lanes), jnp.float32),
            pltpu.VMEM((sub, lanes), jnp.float32),
            pltpu.VMEM((sub, lanes), jnp.float32),
        ],
    )
    ml, tot = pl.pallas_call(
        _wsc_kernel,
        grid_spec=grid_spec,
        out_shape=[
            jax.ShapeDtypeStruct((1,), jnp.float32),
            jax.ShapeDtypeStruct((1,), jnp.float32),
        ],
    )(flat_t, flat_m, flat_in, *([sim3] * K))
    return ml[0], tot[0]


def kernel(input, target, mask, Sim_Matrix):
    b, t, v = input.shape
    flat_in = input.reshape(b * t, SUB, v // SUB)
    flat_t = target[:, :t].reshape(-1)
    flat_m = mask[:, :t].reshape(-1)
    sim3 = Sim_Matrix.reshape(v, SUB, v // SUB)
    return _run(flat_in, flat_t, flat_m, sim3)


# speedup vs baseline: 7.5982x; 2.2402x over previous
import jax
import jax.numpy as jnp
from jax.experimental import pallas as pl
from jax.experimental.pallas import tpu as pltpu

ALPHA = 0.7
TAU_WORD = 0.1
TOK = 64
SUB = 8


def _sim_copy(sim_hbm, sim_buf, sem, tgt_ref, slot, step, k):
    row = tgt_ref[step * TOK + k]
    return pltpu.make_async_copy(
        sim_hbm.at[pl.ds(row, 1), :],
        sim_buf.at[slot, pl.ds(k, 1), :],
        sem.at[slot],
    )


def _wsc_kernel(
    tgt_ref, in_ref, mask_ref, tgt2_ref, sim_hbm,
    out_ref, sim_buf, pr_acc, ss_acc, smem_acc, sem,
):
    i = pl.program_id(0)
    n = pl.num_programs(0)
    v = in_ref.shape[-1]
    slot = jax.lax.rem(i, 2)

    @pl.when(i == 0)
    def _prologue():
        smem_acc[0] = 0.0
        smem_acc[1] = 0.0
        pr_acc[...] = jnp.zeros_like(pr_acc)
        ss_acc[...] = jnp.zeros_like(ss_acc)
        for k in range(TOK):
            _sim_copy(sim_hbm, sim_buf, sem, tgt_ref, 0, 0, k).start()

    @pl.when(i + 1 < n)
    def _prefetch():
        nxt = jax.lax.rem(i + 1, 2)
        for k in range(TOK):
            _sim_copy(sim_hbm, sim_buf, sem, tgt_ref, nxt, i + 1, k).start()

    for k in range(TOK):
        _sim_copy(sim_hbm, sim_buf, sem, tgt_ref, slot, i, k).wait()

    col_iota = jax.lax.broadcasted_iota(jnp.int32, (SUB, v), 1)
    ml_part = jnp.zeros((SUB, 1), jnp.float32)
    for j in range(TOK // SUB):
        sim_t = sim_buf[slot, pl.ds(j * SUB, SUB), :]
        in_t = in_ref[pl.ds(j * SUB, SUB), :]
        m_t = mask_ref[pl.ds(j * SUB, SUB), :]
        tgt_t = tgt2_ref[pl.ds(j * SUB, SUB), :]
        smooth = jnp.exp((sim_t - 1.0) * (1.0 / TAU_WORD))
        t = smooth * m_t
        ss_acc[...] += t
        pr_acc[...] += in_t * t
        hit = col_iota == tgt_t
        ml_part += jnp.sum(jnp.where(hit, in_t, 0.0), axis=1, keepdims=True) * m_t
    smem_acc[0] += jnp.sum(mask_ref[...])
    smem_acc[1] += jnp.sum(ml_part)

    @pl.when(i == n - 1)
    def _fin():
        ml = -smem_acc[1] / smem_acc[0]
        smooth_loss = -jnp.sum(pr_acc[...]) / jnp.sum(ss_acc[...])
        out_ref[0] = ml
        out_ref[1] = ALPHA * smooth_loss + (1.0 - ALPHA) * ml


@jax.jit
def _run(flat_in, flat_t, mask2, tgt2, Sim_Matrix):
    n, v = flat_in.shape
    grid_spec = pltpu.PrefetchScalarGridSpec(
        num_scalar_prefetch=1,
        grid=(n // TOK,),
        in_specs=[
            pl.BlockSpec((TOK, v), lambda i, tgt: (i, 0)),
            pl.BlockSpec((TOK, 1), lambda i, tgt: (i, 0)),
            pl.BlockSpec((TOK, 1), lambda i, tgt: (i, 0)),
            pl.BlockSpec(memory_space=pltpu.HBM),
        ],
        out_specs=pl.BlockSpec(memory_space=pltpu.SMEM),
        scratch_shapes=[
            pltpu.VMEM((2, TOK, v), jnp.float32),
            pltpu.VMEM((SUB, v), jnp.float32),
            pltpu.VMEM((SUB, v), jnp.float32),
            pltpu.SMEM((2,), jnp.float32),
            pltpu.SemaphoreType.DMA((2,)),
        ],
    )
    out = pl.pallas_call(
        _wsc_kernel,
        grid_spec=grid_spec,
        out_shape=jax.ShapeDtypeStruct((2,), jnp.float32),
    )(flat_t, flat_in, mask2, tgt2, Sim_Matrix)
    return out[0], out[1]


def kernel(input, target, mask, Sim_Matrix):
    b, t, v = input.shape
    flat_in = input.reshape(b * t, v)
    flat_t = target[:, :t].reshape(-1)
    mask2 = mask[:, :t].reshape(-1, 1)
    tgt2 = flat_t.reshape(-1, 1)
    return _run(flat_in, flat_t, mask2, tgt2, Sim_Matrix)
